# baseline (speedup 1.0000x reference)
import functools

import jax
import jax.numpy as jnp
from jax import lax
from jax.experimental import pallas as pl
from jax.experimental.pallas import tpu as pltpu
from jax.experimental.pallas import tpu_sc as plsc

N = 10000
E = 320000
NFEAT = 128
NHID = 128
NCLASS = 64

CHUNK = 128
NSC = 2
NTILES = 16
NW = NSC * NTILES
NP = 10240
RPT = NP // NTILES
DEG_W = 128
BLKCH = 16
BLOCKS = 5
NBLK = NW * BLOCKS
BLK0 = 10
BLK1 = 0
TPT = BLOCKS * BLKCH
EPAD = NW * TPT * CHUNK
NBUF = 2


@functools.cache
def _mesh():
    return plsc.VectorSubcoreMesh(core_axis_name="c", subcore_axis_name="s")


@functools.cache
def _deg_kernel_fn():
    @functools.partial(
        pl.kernel,
        out_type=jax.ShapeDtypeStruct((NSC * NP, DEG_W), jnp.float32),
        mesh=_mesh(),
        scratch_types=[
            pltpu.VMEM((BLKCH, CHUNK), jnp.int32),
            pltpu.VMEM((CHUNK, DEG_W), jnp.float32),
            pltpu.VMEM_SHARED((NP, DEG_W), jnp.float32),
            pltpu.SemaphoreType.DMA,
        ],
    )
    def _deg_kernel(dst_hbm, ones_hbm, zeros_hbm, out_hbm, ibd, ones_v, acc,
                    ssem):
        c = lax.axis_index("c")
        s = lax.axis_index("s")
        wid = s * NSC + c
        r0 = s * RPT
        pltpu.sync_copy(ones_hbm, ones_v)
        pltpu.sync_copy(zeros_hbm.at[pl.ds(r0, RPT)], acc.at[pl.ds(r0, RPT)])
        plsc.subcore_barrier()

        def block(blk, carry):
            bi = wid * BLOCKS + blk
            pltpu.sync_copy(dst_hbm.at[bi], ibd)
            descs = [pltpu.async_copy(ones_v, acc.at[ibd.at[k]], ssem,
                                      add=True)
                     for k in range(BLKCH)]
            for dsc in descs:
                dsc.wait()
            return carry

        lax.fori_loop(0, BLOCKS, block, 0)
        plsc.subcore_barrier()
        pltpu.sync_copy(acc.at[pl.ds(r0, RPT)],
                        out_hbm.at[pl.ds(c * NP + r0, RPT)])

    return _deg_kernel


@functools.cache
def _make_agg(F):
    @functools.partial(
        pl.kernel,
        out_type=jax.ShapeDtypeStruct((NSC * NP, F), jnp.float32),
        mesh=_mesh(),
        scratch_types=[
            pltpu.VMEM((CHUNK,), jnp.int32),
            pltpu.VMEM((CHUNK,), jnp.int32),
            pltpu.VMEM((CHUNK, F), jnp.float32),
            pltpu.VMEM_SHARED((NP, F), jnp.float32),
            pltpu.SemaphoreType.DMA,
        ],
    )
    def agg(y_hbm, src_hbm, dst_hbm, zeros_hbm, out_hbm,
            sidx, didx, rows, acc, gsem):
        c = lax.axis_index("c")
        s = lax.axis_index("s")
        wid = s * NSC + c
        r0 = s * RPT
        pltpu.sync_copy(zeros_hbm.at[pl.ds(r0, RPT)], acc.at[pl.ds(r0, RPT)])
        plsc.subcore_barrier()

        def body(i, carry):
            j = wid + NW * i
            pltpu.sync_copy(src_hbm.at[j], sidx)
            pltpu.sync_copy(dst_hbm.at[j], didx)
            pltpu.async_copy(y_hbm.at[sidx], rows, gsem).wait()
            pltpu.sync_copy(rows, acc.at[didx], add=True)
            return carry

        lax.fori_loop(0, TPT, body, 0)
        plsc.subcore_barrier()
        pltpu.sync_copy(acc.at[pl.ds(r0, RPT)],
                        out_hbm.at[pl.ds(c * NP + r0, RPT)])

    return agg


RBLK = 1000


def _deg_d(degp):
    deg = degp[0, :, 0] + degp[1, :, 0] + 1.0
    return lax.rsqrt(deg)


def _t1_body(x_ref, degp_ref, w_ref, o_ref):
    d = _deg_d(degp_ref[...])
    o_ref[...] = jnp.dot(x_ref[...], w_ref[...],
                         preferred_element_type=jnp.float32) * d[:, None]


def _tmid_body(p_ref, y_ref, degp_ref, b_ref, w_ref, o_ref):
    d = _deg_d(degp_ref[...])
    p = p_ref[0] + p_ref[1] + y_ref[...]
    h = jnp.maximum(p * d[:, None] + b_ref[...], 0.0)
    o_ref[...] = jnp.dot(h, w_ref[...],
                         preferred_element_type=jnp.float32) * d[:, None]


def _tout_body(p_ref, y_ref, degp_ref, b_ref, o_ref):
    d = _deg_d(degp_ref[...])
    p = (p_ref[0] + p_ref[1] + y_ref[...])[:, :NCLASS]
    o_ref[...] = p * d[:, None] + b_ref[...]


def _t1(x, degp, W):
    return pl.pallas_call(
        _t1_body,
        grid=(N // RBLK,),
        in_specs=[
            pl.BlockSpec((RBLK, NFEAT), lambda i: (i, 0)),
            pl.BlockSpec((NSC, RBLK, DEG_W), lambda i: (0, i, 0)),
            pl.BlockSpec((NFEAT, NHID), lambda i: (0, 0)),
        ],
        out_specs=pl.BlockSpec((RBLK, NHID), lambda i: (i, 0)),
        out_shape=jax.ShapeDtypeStruct((N, NHID), jnp.float32),
    )(x, degp, W)


def _tmid(p, y, degp, b, W, fout):
    return pl.pallas_call(
        _tmid_body,
        grid=(N // RBLK,),
        in_specs=[
            pl.BlockSpec((NSC, RBLK, NHID), lambda i: (0, i, 0)),
            pl.BlockSpec((RBLK, NHID), lambda i: (i, 0)),
            pl.BlockSpec((NSC, RBLK, DEG_W), lambda i: (0, i, 0)),
            pl.BlockSpec((1, NHID), lambda i: (0, 0)),
            pl.BlockSpec((NHID, fout), lambda i: (0, 0)),
        ],
        out_specs=pl.BlockSpec((RBLK, fout), lambda i: (i, 0)),
        out_shape=jax.ShapeDtypeStruct((N, fout), jnp.float32),
    )(p, y, degp, b, W)


def _tout(p, y, degp, b):
    return pl.pallas_call(
        _tout_body,
        grid=(N // RBLK,),
        in_specs=[
            pl.BlockSpec((NSC, RBLK, NHID), lambda i: (0, i, 0)),
            pl.BlockSpec((RBLK, NHID), lambda i: (i, 0)),
            pl.BlockSpec((NSC, RBLK, DEG_W), lambda i: (0, i, 0)),
            pl.BlockSpec((1, NCLASS), lambda i: (0, 0)),
        ],
        out_specs=pl.BlockSpec((RBLK, NCLASS), lambda i: (i, 0)),
        out_shape=jax.ShapeDtypeStruct((N, NCLASS), jnp.float32),
    )(p, y, degp, b)


def kernel(x, adj, W1, b1, Wx, bx, W2, b2):
    pad_dst = N + jnp.arange(EPAD - E, dtype=jnp.int32) % (NP - N)
    src_flat = jnp.concatenate(
        [adj[0].astype(jnp.int32), jnp.zeros((EPAD - E,), jnp.int32)])
    dst_flat = jnp.concatenate([adj[1].astype(jnp.int32), pad_dst])
    src = src_flat.reshape(NW * TPT, CHUNK)
    dst = dst_flat.reshape(NW * TPT, CHUNK)
    dst3 = dst_flat.reshape(NW * BLOCKS, BLKCH, CHUNK)

    ones8 = jnp.ones((CHUNK, DEG_W), jnp.float32)
    zeros8 = jnp.zeros((NP, DEG_W), jnp.float32)
    zeros128 = jnp.zeros((NP, NHID), jnp.float32)
    W2p = jnp.concatenate([W2, jnp.zeros((NHID, NHID - NCLASS), jnp.float32)],
                          axis=1)

    degp = _deg_kernel_fn()(dst3, ones8, zeros8).reshape(NSC, NP, DEG_W)

    y1 = _t1(x, degp, W1)
    p1 = _make_agg(NHID)(y1, src, dst, zeros128).reshape(NSC, NP, NHID)
    y2 = _tmid(p1, y1, degp, b1.reshape(1, NHID), Wx, NHID)
    p2 = _make_agg(NHID)(y2, src, dst, zeros128).reshape(NSC, NP, NHID)
    y3 = _tmid(p2, y2, degp, bx.reshape(1, NHID), W2p, NHID)
    q = _make_agg(NHID)(y3, src, dst, zeros128).reshape(NSC, NP, NHID)
    return _tout(q, y3, degp, b2.reshape(1, NCLASS))


# speedup vs baseline: 3.3549x; 3.3549x over previous
import functools

import jax
import jax.numpy as jnp
from jax import lax
from jax.experimental import pallas as pl
from jax.experimental.pallas import tpu as pltpu
from jax.experimental.pallas import tpu_sc as plsc

N = 10000
E = 320000
NFEAT = 128
NHID = 128
NCLASS = 64

CHUNK = 128
NSC = 2
NTILES = 16
NW = NSC * NTILES
NP = 10240
RPT = NP // NTILES
DEG_W = 128
BLKCH = 16
BLOCKS = 5
NBLK = NW * BLOCKS
BLK0 = 10
BLK1 = 0
TPT = BLOCKS * BLKCH
EPAD = NW * TPT * CHUNK
NBUF = 2


@functools.cache
def _mesh():
    return plsc.VectorSubcoreMesh(core_axis_name="c", subcore_axis_name="s")


@functools.cache
def _deg_kernel_fn():
    @functools.partial(
        pl.kernel,
        out_type=jax.ShapeDtypeStruct((NSC * NP, DEG_W), jnp.float32),
        mesh=_mesh(),
        scratch_types=[
            pltpu.VMEM((BLKCH, CHUNK), jnp.int32),
            pltpu.VMEM((CHUNK, DEG_W), jnp.float32),
            pltpu.VMEM_SHARED((NP, DEG_W), jnp.float32),
            pltpu.SemaphoreType.DMA,
        ],
    )
    def _deg_kernel(dst_hbm, ones_hbm, zeros_hbm, out_hbm, ibd, ones_v, acc,
                    ssem):
        c = lax.axis_index("c")
        s = lax.axis_index("s")
        wid = s * NSC + c
        r0 = s * RPT
        pltpu.sync_copy(ones_hbm, ones_v)
        pltpu.sync_copy(zeros_hbm.at[pl.ds(r0, RPT)], acc.at[pl.ds(r0, RPT)])
        plsc.subcore_barrier()

        def block(blk, carry):
            bi = wid * BLOCKS + blk
            pltpu.sync_copy(dst_hbm.at[bi], ibd)
            descs = [pltpu.async_copy(ones_v, acc.at[ibd.at[k]], ssem,
                                      add=True)
                     for k in range(BLKCH)]
            for dsc in descs:
                dsc.wait()
            return carry

        lax.fori_loop(0, BLOCKS, block, 0)
        plsc.subcore_barrier()
        pltpu.sync_copy(acc.at[pl.ds(r0, RPT)],
                        out_hbm.at[pl.ds(c * NP + r0, RPT)])

    return _deg_kernel


@functools.cache
def _make_agg(F):
    @functools.partial(
        pl.kernel,
        out_type=jax.ShapeDtypeStruct((NSC * NP, F), jnp.float32),
        mesh=_mesh(),
        scratch_types=[
            pltpu.VMEM((BLKCH, CHUNK), jnp.int32),
            pltpu.VMEM((BLKCH, CHUNK), jnp.int32),
            pltpu.VMEM((CHUNK, F), jnp.float32),
            pltpu.VMEM((CHUNK, F), jnp.float32),
            pltpu.VMEM_SHARED((NP, F), jnp.float32),
            pltpu.SemaphoreType.DMA,
        ],
    )
    def agg(y_hbm, src_hbm, dst_hbm, zeros_hbm, out_hbm,
            ibs, ibd, rows0, rows1, acc, gsem):
        c = lax.axis_index("c")
        s = lax.axis_index("s")
        wid = s * NSC + c
        r0 = s * RPT
        pltpu.sync_copy(zeros_hbm.at[pl.ds(r0, RPT)], acc.at[pl.ds(r0, RPT)])
        plsc.subcore_barrier()

        def slot(k):
            return rows0 if k % 2 == 0 else rows1

        def block(blk, carry):
            bi = wid * BLOCKS + blk
            pltpu.sync_copy(src_hbm.at[bi], ibs)
            pltpu.sync_copy(dst_hbm.at[bi], ibd)
            gd = [pltpu.async_copy(y_hbm.at[ibs.at[0]], slot(0), gsem)]
            for k in range(BLKCH):
                if k + 1 < BLKCH:
                    gd.append(pltpu.async_copy(y_hbm.at[ibs.at[k + 1]],
                                               slot(k + 1), gsem))
                gd[k].wait()
                pltpu.sync_copy(slot(k), acc.at[ibd.at[k]], add=True)
            return carry

        lax.fori_loop(0, BLOCKS, block, 0)
        plsc.subcore_barrier()
        pltpu.sync_copy(acc.at[pl.ds(r0, RPT)],
                        out_hbm.at[pl.ds(c * NP + r0, RPT)])

    return agg


RBLK = 1000


def _deg_d(degp):
    deg = degp[0, :, 0] + degp[1, :, 0] + 1.0
    return lax.rsqrt(deg)


def _t1_body(x_ref, degp_ref, w_ref, o_ref):
    d = _deg_d(degp_ref[...])
    o_ref[...] = jnp.dot(x_ref[...], w_ref[...],
                         preferred_element_type=jnp.float32) * d[:, None]


def _tmid_body(p_ref, y_ref, degp_ref, b_ref, w_ref, o_ref):
    d = _deg_d(degp_ref[...])
    p = p_ref[0] + p_ref[1] + y_ref[...]
    h = jnp.maximum(p * d[:, None] + b_ref[...], 0.0)
    o_ref[...] = jnp.dot(h, w_ref[...],
                         preferred_element_type=jnp.float32) * d[:, None]


def _tout_body(p_ref, y_ref, degp_ref, b_ref, o_ref):
    d = _deg_d(degp_ref[...])
    p = (p_ref[0] + p_ref[1] + y_ref[...])[:, :NCLASS]
    o_ref[...] = p * d[:, None] + b_ref[...]


def _t1(x, degp, W):
    return pl.pallas_call(
        _t1_body,
        grid=(N // RBLK,),
        in_specs=[
            pl.BlockSpec((RBLK, NFEAT), lambda i: (i, 0)),
            pl.BlockSpec((NSC, RBLK, DEG_W), lambda i: (0, i, 0)),
            pl.BlockSpec((NFEAT, NHID), lambda i: (0, 0)),
        ],
        out_specs=pl.BlockSpec((RBLK, NHID), lambda i: (i, 0)),
        out_shape=jax.ShapeDtypeStruct((N, NHID), jnp.float32),
    )(x, degp, W)


def _tmid(p, y, degp, b, W, fout):
    return pl.pallas_call(
        _tmid_body,
        grid=(N // RBLK,),
        in_specs=[
            pl.BlockSpec((NSC, RBLK, NHID), lambda i: (0, i, 0)),
            pl.BlockSpec((RBLK, NHID), lambda i: (i, 0)),
            pl.BlockSpec((NSC, RBLK, DEG_W), lambda i: (0, i, 0)),
            pl.BlockSpec((1, NHID), lambda i: (0, 0)),
            pl.BlockSpec((NHID, fout), lambda i: (0, 0)),
        ],
        out_specs=pl.BlockSpec((RBLK, fout), lambda i: (i, 0)),
        out_shape=jax.ShapeDtypeStruct((N, fout), jnp.float32),
    )(p, y, degp, b, W)


def _tout(p, y, degp, b):
    return pl.pallas_call(
        _tout_body,
        grid=(N // RBLK,),
        in_specs=[
            pl.BlockSpec((NSC, RBLK, NHID), lambda i: (0, i, 0)),
            pl.BlockSpec((RBLK, NHID), lambda i: (i, 0)),
            pl.BlockSpec((NSC, RBLK, DEG_W), lambda i: (0, i, 0)),
            pl.BlockSpec((1, NCLASS), lambda i: (0, 0)),
        ],
        out_specs=pl.BlockSpec((RBLK, NCLASS), lambda i: (i, 0)),
        out_shape=jax.ShapeDtypeStruct((N, NCLASS), jnp.float32),
    )(p, y, degp, b)


def kernel(x, adj, W1, b1, Wx, bx, W2, b2):
    npad = EPAD - E
    pad_src = jnp.arange(npad, dtype=jnp.int32) % N
    pad_dst = N + jnp.arange(npad, dtype=jnp.int32) % (NP - N)
    src = jnp.concatenate(
        [adj[0].astype(jnp.int32), pad_src]).reshape(NW * BLOCKS, BLKCH, CHUNK)
    dst = jnp.concatenate(
        [adj[1].astype(jnp.int32), pad_dst]).reshape(NW * BLOCKS, BLKCH, CHUNK)

    ones8 = jnp.ones((CHUNK, DEG_W), jnp.float32)
    zeros8 = jnp.zeros((NP, DEG_W), jnp.float32)
    zeros128 = jnp.zeros((NP, NHID), jnp.float32)
    W2p = jnp.concatenate([W2, jnp.zeros((NHID, NHID - NCLASS), jnp.float32)],
                          axis=1)

    degp = _deg_kernel_fn()(dst, ones8, zeros8).reshape(NSC, NP, DEG_W)

    y1 = _t1(x, degp, W1)
    p1 = _make_agg(NHID)(y1, src, dst, zeros128).reshape(NSC, NP, NHID)
    y2 = _tmid(p1, y1, degp, b1.reshape(1, NHID), Wx, NHID)
    p2 = _make_agg(NHID)(y2, src, dst, zeros128).reshape(NSC, NP, NHID)
    y3 = _tmid(p2, y2, degp, bx.reshape(1, NHID), W2p, NHID)
    q = _make_agg(NHID)(y3, src, dst, zeros128).reshape(NSC, NP, NHID)
    return _tout(q, y3, degp, b2.reshape(1, NCLASS))


# speedup vs baseline: 3.3965x; 1.0124x over previous
import functools

import jax
import jax.numpy as jnp
from jax import lax
from jax.experimental import pallas as pl
from jax.experimental.pallas import tpu as pltpu
from jax.experimental.pallas import tpu_sc as plsc

N = 10000
E = 320000
NFEAT = 128
NHID = 128
NCLASS = 64

CHUNK = 64
NSC = 2
NTILES = 16
NW = NSC * NTILES
NP = 10240
RPT = NP // NTILES
DEG_W = 128
BLKCH = 16
BLOCKS = 10
NBLK = NW * BLOCKS
BLK0 = 10
BLK1 = 0
TPT = BLOCKS * BLKCH
EPAD = NW * TPT * CHUNK
NBUF = 2


@functools.cache
def _mesh():
    return plsc.VectorSubcoreMesh(core_axis_name="c", subcore_axis_name="s")


@functools.cache
def _deg_kernel_fn():
    @functools.partial(
        pl.kernel,
        out_type=jax.ShapeDtypeStruct((NSC * NP, DEG_W), jnp.float32),
        mesh=_mesh(),
        scratch_types=[
            pltpu.VMEM((BLKCH, CHUNK), jnp.int32),
            pltpu.VMEM((CHUNK, DEG_W), jnp.float32),
            pltpu.VMEM_SHARED((NP, DEG_W), jnp.float32),
            pltpu.SemaphoreType.DMA,
        ],
    )
    def _deg_kernel(dst_hbm, ones_hbm, zeros_hbm, out_hbm, ibd, ones_v, acc,
                    ssem):
        c = lax.axis_index("c")
        s = lax.axis_index("s")
        wid = s * NSC + c
        r0 = s * RPT
        pltpu.sync_copy(ones_hbm, ones_v)
        pltpu.sync_copy(zeros_hbm.at[pl.ds(r0, RPT)], acc.at[pl.ds(r0, RPT)])
        plsc.subcore_barrier()

        def block(blk, carry):
            bi = wid * BLOCKS + blk
            pltpu.sync_copy(dst_hbm.at[bi], ibd)
            descs = [pltpu.async_copy(ones_v, acc.at[ibd.at[k]], ssem,
                                      add=True)
                     for k in range(BLKCH)]
            for dsc in descs:
                dsc.wait()
            return carry

        lax.fori_loop(0, BLOCKS, block, 0)
        plsc.subcore_barrier()
        pltpu.sync_copy(acc.at[pl.ds(r0, RPT)],
                        out_hbm.at[pl.ds(c * NP + r0, RPT)])

    return _deg_kernel


@functools.cache
def _make_agg(F):
    @functools.partial(
        pl.kernel,
        out_type=jax.ShapeDtypeStruct((NSC * NP, F), jnp.float32),
        mesh=_mesh(),
        scratch_types=[
            pltpu.VMEM((BLKCH, CHUNK), jnp.int32),
            pltpu.VMEM((BLKCH, CHUNK), jnp.int32),
            pltpu.VMEM((CHUNK, F), jnp.float32),
            pltpu.VMEM((CHUNK, F), jnp.float32),
            pltpu.VMEM((CHUNK, F), jnp.float32),
            pltpu.VMEM((CHUNK, F), jnp.float32),
            pltpu.VMEM_SHARED((NP, F), jnp.float32),
            pltpu.SemaphoreType.DMA,
        ],
    )
    def agg(y_hbm, src_hbm, dst_hbm, zeros_hbm, out_hbm,
            ibs, ibd, rows0, rows1, rows2, rows3, acc, gsem):
        c = lax.axis_index("c")
        s = lax.axis_index("s")
        wid = s * NSC + c
        r0 = s * RPT
        pltpu.sync_copy(zeros_hbm.at[pl.ds(r0, RPT)], acc.at[pl.ds(r0, RPT)])
        plsc.subcore_barrier()

        slots = [rows0, rows1, rows2, rows3]

        def slot(k):
            return slots[k % 4]

        def block(blk, carry):
            bi = wid * BLOCKS + blk
            pltpu.sync_copy(src_hbm.at[bi], ibs)
            pltpu.sync_copy(dst_hbm.at[bi], ibd)
            gd = [pltpu.async_copy(y_hbm.at[ibs.at[k]], slot(k), gsem)
                  for k in range(3)]
            for k in range(BLKCH):
                if k + 3 < BLKCH:
                    gd.append(pltpu.async_copy(y_hbm.at[ibs.at[k + 3]],
                                               slot(k + 3), gsem))
                gd[k].wait()
                pltpu.sync_copy(slot(k), acc.at[ibd.at[k]], add=True)
            return carry

        lax.fori_loop(0, BLOCKS, block, 0)
        plsc.subcore_barrier()
        pltpu.sync_copy(acc.at[pl.ds(r0, RPT)],
                        out_hbm.at[pl.ds(c * NP + r0, RPT)])

    return agg


RBLK = 1000


def _deg_d(degp):
    deg = degp[0, :, 0] + degp[1, :, 0] + 1.0
    return lax.rsqrt(deg)


def _t1_body(x_ref, degp_ref, w_ref, o_ref):
    d = _deg_d(degp_ref[...])
    o_ref[...] = jnp.dot(x_ref[...], w_ref[...],
                         preferred_element_type=jnp.float32) * d[:, None]


def _tmid_body(p_ref, y_ref, degp_ref, b_ref, w_ref, o_ref):
    d = _deg_d(degp_ref[...])
    p = p_ref[0] + p_ref[1] + y_ref[...]
    h = jnp.maximum(p * d[:, None] + b_ref[...], 0.0)
    o_ref[...] = jnp.dot(h, w_ref[...],
                         preferred_element_type=jnp.float32) * d[:, None]


def _tout_body(p_ref, y_ref, degp_ref, b_ref, o_ref):
    d = _deg_d(degp_ref[...])
    p = (p_ref[0] + p_ref[1] + y_ref[...])[:, :NCLASS]
    o_ref[...] = p * d[:, None] + b_ref[...]


def _t1(x, degp, W):
    return pl.pallas_call(
        _t1_body,
        grid=(N // RBLK,),
        in_specs=[
            pl.BlockSpec((RBLK, NFEAT), lambda i: (i, 0)),
            pl.BlockSpec((NSC, RBLK, DEG_W), lambda i: (0, i, 0)),
            pl.BlockSpec((NFEAT, NHID), lambda i: (0, 0)),
        ],
        out_specs=pl.BlockSpec((RBLK, NHID), lambda i: (i, 0)),
        out_shape=jax.ShapeDtypeStruct((N, NHID), jnp.float32),
    )(x, degp, W)


def _tmid(p, y, degp, b, W, fout):
    return pl.pallas_call(
        _tmid_body,
        grid=(N // RBLK,),
        in_specs=[
            pl.BlockSpec((NSC, RBLK, NHID), lambda i: (0, i, 0)),
            pl.BlockSpec((RBLK, NHID), lambda i: (i, 0)),
            pl.BlockSpec((NSC, RBLK, DEG_W), lambda i: (0, i, 0)),
            pl.BlockSpec((1, NHID), lambda i: (0, 0)),
            pl.BlockSpec((NHID, fout), lambda i: (0, 0)),
        ],
        out_specs=pl.BlockSpec((RBLK, fout), lambda i: (i, 0)),
        out_shape=jax.ShapeDtypeStruct((N, fout), jnp.float32),
    )(p, y, degp, b, W)


def _tout(p, y, degp, b):
    return pl.pallas_call(
        _tout_body,
        grid=(N // RBLK,),
        in_specs=[
            pl.BlockSpec((NSC, RBLK, NHID), lambda i: (0, i, 0)),
            pl.BlockSpec((RBLK, NHID), lambda i: (i, 0)),
            pl.BlockSpec((NSC, RBLK, DEG_W), lambda i: (0, i, 0)),
            pl.BlockSpec((1, NCLASS), lambda i: (0, 0)),
        ],
        out_specs=pl.BlockSpec((RBLK, NCLASS), lambda i: (i, 0)),
        out_shape=jax.ShapeDtypeStruct((N, NCLASS), jnp.float32),
    )(p, y, degp, b)


def kernel(x, adj, W1, b1, Wx, bx, W2, b2):
    npad = EPAD - E
    pad_src = jnp.arange(npad, dtype=jnp.int32) % N
    pad_dst = N + jnp.arange(npad, dtype=jnp.int32) % (NP - N)
    src = jnp.concatenate(
        [adj[0].astype(jnp.int32), pad_src]).reshape(NW * BLOCKS, BLKCH, CHUNK)
    dst = jnp.concatenate(
        [adj[1].astype(jnp.int32), pad_dst]).reshape(NW * BLOCKS, BLKCH, CHUNK)

    ones8 = jnp.ones((CHUNK, DEG_W), jnp.float32)
    zeros8 = jnp.zeros((NP, DEG_W), jnp.float32)
    zeros128 = jnp.zeros((NP, NHID), jnp.float32)
    W2p = jnp.concatenate([W2, jnp.zeros((NHID, NHID - NCLASS), jnp.float32)],
                          axis=1)

    degp = _deg_kernel_fn()(dst, ones8, zeros8).reshape(NSC, NP, DEG_W)

    y1 = _t1(x, degp, W1)
    p1 = _make_agg(NHID)(y1, src, dst, zeros128).reshape(NSC, NP, NHID)
    y2 = _tmid(p1, y1, degp, b1.reshape(1, NHID), Wx, NHID)
    p2 = _make_agg(NHID)(y2, src, dst, zeros128).reshape(NSC, NP, NHID)
    y3 = _tmid(p2, y2, degp, bx.reshape(1, NHID), W2p, NHID)
    q = _make_agg(NHID)(y3, src, dst, zeros128).reshape(NSC, NP, NHID)
    return _tout(q, y3, degp, b2.reshape(1, NCLASS))


# speedup vs baseline: 3.4518x; 1.0163x over previous
import functools

import jax
import jax.numpy as jnp
from jax import lax
from jax.experimental import pallas as pl
from jax.experimental.pallas import tpu as pltpu
from jax.experimental.pallas import tpu_sc as plsc

N = 10000
E = 320000
NFEAT = 128
NHID = 128
NCLASS = 64

CHUNK = 64
NSC = 2
NTILES = 16
NW = NSC * NTILES
NP = 10240
RPT = NP // NTILES
DEG_W = 128
BLKCH = 16
BLOCKS = 10
NBLK = NW * BLOCKS
BLK0 = 10
BLK1 = 0
TPT = BLOCKS * BLKCH
EPAD = NW * TPT * CHUNK
NBUF = 2


@functools.cache
def _mesh():
    return plsc.VectorSubcoreMesh(core_axis_name="c", subcore_axis_name="s")


@functools.cache
def _deg_kernel_fn():
    @functools.partial(
        pl.kernel,
        out_type=jax.ShapeDtypeStruct((NSC * NP, DEG_W), jnp.float32),
        mesh=_mesh(),
        scratch_types=[
            pltpu.VMEM((BLKCH, CHUNK), jnp.int32),
            pltpu.VMEM((CHUNK, DEG_W), jnp.float32),
            pltpu.VMEM_SHARED((NP, DEG_W), jnp.float32),
            pltpu.SemaphoreType.DMA,
        ],
    )
    def _deg_kernel(dst_hbm, ones_hbm, zeros_hbm, out_hbm, ibd, ones_v, acc,
                    ssem):
        c = lax.axis_index("c")
        s = lax.axis_index("s")
        wid = s * NSC + c
        r0 = s * RPT
        pltpu.sync_copy(ones_hbm, ones_v)
        pltpu.sync_copy(zeros_hbm.at[pl.ds(r0, RPT)], acc.at[pl.ds(r0, RPT)])
        plsc.subcore_barrier()

        def block(blk, carry):
            bi = wid * BLOCKS + blk
            pltpu.sync_copy(dst_hbm.at[bi], ibd)
            descs = [pltpu.async_copy(ones_v, acc.at[ibd.at[k]], ssem,
                                      add=True)
                     for k in range(BLKCH)]
            for dsc in descs:
                dsc.wait()
            return carry

        lax.fori_loop(0, BLOCKS, block, 0)
        plsc.subcore_barrier()
        pltpu.sync_copy(acc.at[pl.ds(r0, RPT)],
                        out_hbm.at[pl.ds(c * NP + r0, RPT)])

    return _deg_kernel


@functools.cache
def _make_agg(F):
    @functools.partial(
        pl.kernel,
        out_type=jax.ShapeDtypeStruct((NSC * NP, F), jnp.float32),
        mesh=_mesh(),
        scratch_types=[
            pltpu.VMEM((BLKCH, CHUNK), jnp.int32),
            pltpu.VMEM((BLKCH, CHUNK), jnp.int32),
            pltpu.VMEM((CHUNK, F), jnp.float32),
            pltpu.VMEM((CHUNK, F), jnp.float32),
            pltpu.VMEM((CHUNK, F), jnp.float32),
            pltpu.VMEM((CHUNK, F), jnp.float32),
            pltpu.VMEM_SHARED((NP, F), jnp.float32),
            pltpu.SemaphoreType.DMA,
        ],
    )
    def agg(y_hbm, src_hbm, dst_hbm, zeros_hbm, out_hbm,
            ibs, ibd, rows0, rows1, rows2, rows3, acc, gsem):
        c = lax.axis_index("c")
        s = lax.axis_index("s")
        wid = s * NSC + c
        r0 = s * RPT
        pltpu.sync_copy(zeros_hbm.at[pl.ds(r0, RPT)], acc.at[pl.ds(r0, RPT)])
        plsc.subcore_barrier()

        slots = [rows0, rows1, rows2, rows3]

        def slot(k):
            return slots[k % 4]

        def block(blk, carry):
            bi = wid * BLOCKS + blk
            pltpu.sync_copy(src_hbm.at[bi], ibs)
            pltpu.sync_copy(dst_hbm.at[bi], ibd)
            gd = [pltpu.async_copy(y_hbm.at[ibs.at[k]], slot(k), gsem)
                  for k in range(3)]
            for k in range(BLKCH):
                if k + 3 < BLKCH:
                    gd.append(pltpu.async_copy(y_hbm.at[ibs.at[k + 3]],
                                               slot(k + 3), gsem))
                gd[k].wait()
                pltpu.sync_copy(slot(k), acc.at[ibd.at[k]], add=True)
            return carry

        lax.fori_loop(0, BLOCKS, block, 0)
        plsc.subcore_barrier()
        pltpu.sync_copy(acc.at[pl.ds(r0, RPT)],
                        out_hbm.at[pl.ds(c * NP + r0, RPT)])

    return agg


RBLK = 2000


def _deg_d(degp):
    deg = degp[0, :, 0] + degp[1, :, 0] + 1.0
    return lax.rsqrt(deg)


def _t1_body(x_ref, degp_ref, w_ref, o_ref):
    d = _deg_d(degp_ref[...])
    o_ref[...] = jnp.dot(x_ref[...], w_ref[...],
                         preferred_element_type=jnp.float32) * d[:, None]


def _tmid_body(p_ref, y_ref, degp_ref, b_ref, w_ref, o_ref):
    d = _deg_d(degp_ref[...])
    p = p_ref[0] + p_ref[1] + y_ref[...]
    h = jnp.maximum(p * d[:, None] + b_ref[...], 0.0)
    o_ref[...] = jnp.dot(h, w_ref[...],
                         preferred_element_type=jnp.float32) * d[:, None]


def _tout_body(p_ref, y_ref, degp_ref, b_ref, o_ref):
    d = _deg_d(degp_ref[...])
    p = (p_ref[0] + p_ref[1] + y_ref[...])[:, :NCLASS]
    o_ref[...] = p * d[:, None] + b_ref[...]


def _t1(x, degp, W):
    return pl.pallas_call(
        _t1_body,
        grid=(N // RBLK,),
        in_specs=[
            pl.BlockSpec((RBLK, NFEAT), lambda i: (i, 0)),
            pl.BlockSpec((NSC, RBLK, DEG_W), lambda i: (0, i, 0)),
            pl.BlockSpec((NFEAT, NHID), lambda i: (0, 0)),
        ],
        out_specs=pl.BlockSpec((RBLK, NHID), lambda i: (i, 0)),
        out_shape=jax.ShapeDtypeStruct((N, NHID), jnp.float32),
    )(x, degp, W)


def _tmid(p, y, degp, b, W, fout):
    return pl.pallas_call(
        _tmid_body,
        grid=(N // RBLK,),
        in_specs=[
            pl.BlockSpec((NSC, RBLK, NHID), lambda i: (0, i, 0)),
            pl.BlockSpec((RBLK, NHID), lambda i: (i, 0)),
            pl.BlockSpec((NSC, RBLK, DEG_W), lambda i: (0, i, 0)),
            pl.BlockSpec((1, NHID), lambda i: (0, 0)),
            pl.BlockSpec((NHID, fout), lambda i: (0, 0)),
        ],
        out_specs=pl.BlockSpec((RBLK, fout), lambda i: (i, 0)),
        out_shape=jax.ShapeDtypeStruct((N, fout), jnp.float32),
    )(p, y, degp, b, W)


def _tout(p, y, degp, b):
    return pl.pallas_call(
        _tout_body,
        grid=(N // RBLK,),
        in_specs=[
            pl.BlockSpec((NSC, RBLK, NHID), lambda i: (0, i, 0)),
            pl.BlockSpec((RBLK, NHID), lambda i: (i, 0)),
            pl.BlockSpec((NSC, RBLK, DEG_W), lambda i: (0, i, 0)),
            pl.BlockSpec((1, NCLASS), lambda i: (0, 0)),
        ],
        out_specs=pl.BlockSpec((RBLK, NCLASS), lambda i: (i, 0)),
        out_shape=jax.ShapeDtypeStruct((N, NCLASS), jnp.float32),
    )(p, y, degp, b)


def kernel(x, adj, W1, b1, Wx, bx, W2, b2):
    npad = EPAD - E
    pad_src = jnp.arange(npad, dtype=jnp.int32) % N
    pad_dst = N + jnp.arange(npad, dtype=jnp.int32) % (NP - N)
    src = jnp.concatenate(
        [adj[0].astype(jnp.int32), pad_src]).reshape(NW * BLOCKS, BLKCH, CHUNK)
    dst = jnp.concatenate(
        [adj[1].astype(jnp.int32), pad_dst]).reshape(NW * BLOCKS, BLKCH, CHUNK)

    ones8 = jnp.ones((CHUNK, DEG_W), jnp.float32)
    zeros8 = jnp.zeros((NP, DEG_W), jnp.float32)
    zeros128 = jnp.zeros((NP, NHID), jnp.float32)
    W2p = jnp.concatenate([W2, jnp.zeros((NHID, NHID - NCLASS), jnp.float32)],
                          axis=1)

    degp = _deg_kernel_fn()(dst, ones8, zeros8).reshape(NSC, NP, DEG_W)

    y1 = _t1(x, degp, W1)
    p1 = _make_agg(NHID)(y1, src, dst, zeros128).reshape(NSC, NP, NHID)
    y2 = _tmid(p1, y1, degp, b1.reshape(1, NHID), Wx, NHID)
    p2 = _make_agg(NHID)(y2, src, dst, zeros128).reshape(NSC, NP, NHID)
    y3 = _tmid(p2, y2, degp, bx.reshape(1, NHID), W2p, NHID)
    q = _make_agg(NHID)(y3, src, dst, zeros128).reshape(NSC, NP, NHID)
    return _tout(q, y3, degp, b2.reshape(1, NCLASS))


# speedup vs baseline: 3.4573x; 1.0016x over previous
import functools

import jax
import jax.numpy as jnp
from jax import lax
from jax.experimental import pallas as pl
from jax.experimental.pallas import tpu as pltpu
from jax.experimental.pallas import tpu_sc as plsc

N = 10000
E = 320000
NFEAT = 128
NHID = 128
NCLASS = 64

CHUNK = 64
NSC = 2
NTILES = 16
NW = NSC * NTILES
NP = 10240
RPT = NP // NTILES
DEG_W = 128
BLKCH = 16
BLOCKS = 10
NBLK = NW * BLOCKS
BLK0 = 10
BLK1 = 0
TPT = BLOCKS * BLKCH
EPAD = NW * TPT * CHUNK
NBUF = 2


@functools.cache
def _mesh():
    return plsc.VectorSubcoreMesh(core_axis_name="c", subcore_axis_name="s")


@functools.cache
def _deg_kernel_fn():
    @functools.partial(
        pl.kernel,
        out_type=jax.ShapeDtypeStruct((NSC * NP, DEG_W), jnp.float32),
        mesh=_mesh(),
        scratch_types=[
            pltpu.VMEM((BLKCH, CHUNK), jnp.int32),
            pltpu.VMEM((CHUNK, DEG_W), jnp.float32),
            pltpu.VMEM_SHARED((NP, DEG_W), jnp.float32),
            pltpu.SemaphoreType.DMA,
        ],
    )
    def _deg_kernel(dst_hbm, ones_hbm, zeros_hbm, out_hbm, ibd, ones_v, acc,
                    ssem):
        c = lax.axis_index("c")
        s = lax.axis_index("s")
        wid = s * NSC + c
        r0 = s * RPT
        pltpu.sync_copy(ones_hbm, ones_v)
        pltpu.sync_copy(zeros_hbm.at[pl.ds(r0, RPT)], acc.at[pl.ds(r0, RPT)])
        plsc.subcore_barrier()

        def block(blk, carry):
            bi = wid * BLOCKS + blk
            pltpu.sync_copy(dst_hbm.at[bi], ibd)
            descs = [pltpu.async_copy(ones_v, acc.at[ibd.at[k]], ssem,
                                      add=True)
                     for k in range(BLKCH)]
            for dsc in descs:
                dsc.wait()
            return carry

        lax.fori_loop(0, BLOCKS, block, 0)
        plsc.subcore_barrier()
        pltpu.sync_copy(acc.at[pl.ds(r0, RPT)],
                        out_hbm.at[pl.ds(c * NP + r0, RPT)])

    return _deg_kernel


@functools.cache
def _make_agg(F):
    @functools.partial(
        pl.kernel,
        out_type=jax.ShapeDtypeStruct((NSC * NP, F), jnp.float32),
        mesh=_mesh(),
        scratch_types=[
            pltpu.VMEM((BLKCH, CHUNK), jnp.int32),
            pltpu.VMEM((BLKCH, CHUNK), jnp.int32),
            pltpu.VMEM((CHUNK, F), jnp.float32),
            pltpu.VMEM((CHUNK, F), jnp.float32),
            pltpu.VMEM((CHUNK, F), jnp.float32),
            pltpu.VMEM((CHUNK, F), jnp.float32),
            pltpu.VMEM_SHARED((NP, F), jnp.float32),
            pltpu.SemaphoreType.DMA,
        ],
    )
    def agg(y_hbm, src_hbm, dst_hbm, zeros_hbm, out_hbm,
            ibs, ibd, rows0, rows1, rows2, rows3, acc, gsem):
        c = lax.axis_index("c")
        s = lax.axis_index("s")
        wid = s * NSC + c
        r0 = s * RPT
        pltpu.sync_copy(zeros_hbm.at[pl.ds(r0, RPT)], acc.at[pl.ds(r0, RPT)])
        plsc.subcore_barrier()

        slots = [rows0, rows1, rows2, rows3]

        def slot(k):
            return slots[k % 4]

        def block(blk, carry):
            bi = wid * BLOCKS + blk
            pltpu.sync_copy(src_hbm.at[bi], ibs)
            pltpu.sync_copy(dst_hbm.at[bi], ibd)
            gd = [pltpu.async_copy(y_hbm.at[ibs.at[k]], slot(k), gsem)
                  for k in range(3)]
            for k in range(BLKCH):
                if k + 3 < BLKCH:
                    gd.append(pltpu.async_copy(y_hbm.at[ibs.at[k + 3]],
                                               slot(k + 3), gsem))
                gd[k].wait()
                pltpu.sync_copy(slot(k), acc.at[ibd.at[k]], add=True)
            return carry

        lax.fori_loop(0, BLOCKS, block, 0)
        plsc.subcore_barrier()
        pltpu.sync_copy(acc.at[pl.ds(r0, RPT)],
                        out_hbm.at[pl.ds(c * NP + r0, RPT)])

    return agg


RBLK = 5000


def _deg_d(degp):
    deg = degp[0, :, 0] + degp[1, :, 0] + 1.0
    return lax.rsqrt(deg)


def _t1_body(x_ref, degp_ref, w_ref, o_ref):
    d = _deg_d(degp_ref[...])
    o_ref[...] = jnp.dot(x_ref[...], w_ref[...],
                         preferred_element_type=jnp.float32) * d[:, None]


def _tmid_body(p_ref, y_ref, degp_ref, b_ref, w_ref, o_ref):
    d = _deg_d(degp_ref[...])
    p = p_ref[0] + p_ref[1] + y_ref[...]
    h = jnp.maximum(p * d[:, None] + b_ref[...], 0.0)
    o_ref[...] = jnp.dot(h, w_ref[...],
                         preferred_element_type=jnp.float32) * d[:, None]


def _tout_body(p_ref, y_ref, degp_ref, b_ref, o_ref):
    d = _deg_d(degp_ref[...])
    p = (p_ref[0] + p_ref[1] + y_ref[...])[:, :NCLASS]
    o_ref[...] = p * d[:, None] + b_ref[...]


def _t1(x, degp, W):
    return pl.pallas_call(
        _t1_body,
        grid=(N // RBLK,),
        in_specs=[
            pl.BlockSpec((RBLK, NFEAT), lambda i: (i, 0)),
            pl.BlockSpec((NSC, RBLK, DEG_W), lambda i: (0, i, 0)),
            pl.BlockSpec((NFEAT, NHID), lambda i: (0, 0)),
        ],
        out_specs=pl.BlockSpec((RBLK, NHID), lambda i: (i, 0)),
        out_shape=jax.ShapeDtypeStruct((N, NHID), jnp.float32),
    )(x, degp, W)


def _tmid(p, y, degp, b, W, fout):
    return pl.pallas_call(
        _tmid_body,
        grid=(N // RBLK,),
        in_specs=[
            pl.BlockSpec((NSC, RBLK, NHID), lambda i: (0, i, 0)),
            pl.BlockSpec((RBLK, NHID), lambda i: (i, 0)),
            pl.BlockSpec((NSC, RBLK, DEG_W), lambda i: (0, i, 0)),
            pl.BlockSpec((1, NHID), lambda i: (0, 0)),
            pl.BlockSpec((NHID, fout), lambda i: (0, 0)),
        ],
        out_specs=pl.BlockSpec((RBLK, fout), lambda i: (i, 0)),
        out_shape=jax.ShapeDtypeStruct((N, fout), jnp.float32),
    )(p, y, degp, b, W)


def _tout(p, y, degp, b):
    return pl.pallas_call(
        _tout_body,
        grid=(N // RBLK,),
        in_specs=[
            pl.BlockSpec((NSC, RBLK, NHID), lambda i: (0, i, 0)),
            pl.BlockSpec((RBLK, NHID), lambda i: (i, 0)),
            pl.BlockSpec((NSC, RBLK, DEG_W), lambda i: (0, i, 0)),
            pl.BlockSpec((1, NCLASS), lambda i: (0, 0)),
        ],
        out_specs=pl.BlockSpec((RBLK, NCLASS), lambda i: (i, 0)),
        out_shape=jax.ShapeDtypeStruct((N, NCLASS), jnp.float32),
    )(p, y, degp, b)


def kernel(x, adj, W1, b1, Wx, bx, W2, b2):
    npad = EPAD - E
    pad_src = jnp.arange(npad, dtype=jnp.int32) % N
    pad_dst = N + jnp.arange(npad, dtype=jnp.int32) % (NP - N)
    src = jnp.concatenate(
        [adj[0].astype(jnp.int32), pad_src]).reshape(NW * BLOCKS, BLKCH, CHUNK)
    dst = jnp.concatenate(
        [adj[1].astype(jnp.int32), pad_dst]).reshape(NW * BLOCKS, BLKCH, CHUNK)

    ones8 = jnp.ones((CHUNK, DEG_W), jnp.float32)
    zeros8 = jnp.zeros((NP, DEG_W), jnp.float32)
    zeros128 = jnp.zeros((NP, NHID), jnp.float32)
    W2p = jnp.concatenate([W2, jnp.zeros((NHID, NHID - NCLASS), jnp.float32)],
                          axis=1)

    degp = _deg_kernel_fn()(dst, ones8, zeros8).reshape(NSC, NP, DEG_W)

    y1 = _t1(x, degp, W1)
    p1 = _make_agg(NHID)(y1, src, dst, zeros128).reshape(NSC, NP, NHID)
    y2 = _tmid(p1, y1, degp, b1.reshape(1, NHID), Wx, NHID)
    p2 = _make_agg(NHID)(y2, src, dst, zeros128).reshape(NSC, NP, NHID)
    y3 = _tmid(p2, y2, degp, bx.reshape(1, NHID), W2p, NHID)
    q = _make_agg(NHID)(y3, src, dst, zeros128).reshape(NSC, NP, NHID)
    return _tout(q, y3, degp, b2.reshape(1, NCLASS))
